# baseline (speedup 1.0000x reference)
import functools

import jax
import jax.numpy as jnp
from jax import lax
from jax.experimental import pallas as pl
from jax.experimental.pallas import tpu as pltpu
from jax.experimental.pallas import tpu_sc as plsc

_PATCH = 32
_THRESHOLD = 0.4
_B = 64
_HW = 512
_HP = _HW // _PATCH
_WP = _HW // _PATCH
_N = _HP * _WP
_L = 16
_NW = 32
_ROWS_PER_W = _B // _NW


_G = 16


def _pool_body(x_ref, p_ref, out_ref):
    v = x_ref[:, 0]
    v1 = jnp.max(v.reshape(_G * _HP, _PATCH, _HW), axis=1)
    ind = (v1 > _THRESHOLD).astype(jnp.bfloat16)
    res = lax.dot_general(
        ind, p_ref[...], (((1,), (0,)), ((), ())),
        preferred_element_type=jnp.float32)
    out_ref[:, 0] = res.reshape(_G, _HP, _WP)


@functools.lru_cache(maxsize=None)
def _build_sort_kernel():
    mesh = plsc.VectorSubcoreMesh(core_axis_name="c", subcore_axis_name="s")
    return functools.partial(
        pl.kernel,
        mesh=mesh,
        compiler_params=pltpu.CompilerParams(needs_layout_passes=False),
        out_type=[
            jax.ShapeDtypeStruct((_B, _N), jnp.int32),
            jax.ShapeDtypeStruct((_B, _N), jnp.int32),
        ],
        scratch_types=[
            pltpu.VMEM((2 * _N,), jnp.float32),
            pltpu.VMEM((_N,), jnp.int32),
            pltpu.VMEM((_N,), jnp.int32),
        ],
    )(_sort_body)


def _sort_body(q_hbm, imap_hbm, inv_hbm, q_v, imap_v, inv_v):
    wid = lax.axis_index("s") * 2 + lax.axis_index("c")
    half = jnp.float32(0.5)
    for rr in range(_ROWS_PER_W):
        row = wid * _ROWS_PER_W + rr
        pltpu.sync_copy(q_hbm.at[row], q_v)

        c0 = jnp.int32(0)
        c1 = jnp.int32(0)
        for i in range(_N // _L):
            a = q_v[pl.ds(i * _L, _L)]
            b = q_v[pl.ds(_N + i * _L, _L)]
            k2 = b > half
            k0 = jnp.logical_and(a > half, jnp.logical_not(k2))
            k1 = jnp.logical_not(jnp.logical_or(k0, k2))
            c0 = c0 + jnp.sum(k0.astype(jnp.int32))
            c1 = c1 + jnp.sum(k1.astype(jnp.int32))

        r0 = jnp.int32(0)
        r1 = c0
        r2 = c0 + c1
        for i in range(_N // _L):
            a = q_v[pl.ds(i * _L, _L)]
            b = q_v[pl.ds(_N + i * _L, _L)]
            k2 = b > half
            k0 = jnp.logical_and(a > half, jnp.logical_not(k2))
            k1 = jnp.logical_not(jnp.logical_or(k0, k2))
            i0 = k0.astype(jnp.int32)
            i1 = k1.astype(jnp.int32)
            i2 = k2.astype(jnp.int32)
            e0 = plsc.cumsum(i0) - i0
            e1 = plsc.cumsum(i1) - i1
            e2 = plsc.cumsum(i2) - i2
            rank = jnp.where(k0, r0 + e0, jnp.where(k2, r2 + e2, r1 + e1))
            pos = lax.iota(jnp.int32, _L) + (i * _L)
            inv_v[pl.ds(i * _L, _L)] = rank
            plsc.store_scatter(imap_v, [rank], pos)
            r0 = r0 + jnp.sum(i0)
            r1 = r1 + jnp.sum(i1)
            r2 = r2 + jnp.sum(i2)

        pltpu.sync_copy(imap_v, imap_hbm.at[row])
        pltpu.sync_copy(inv_v, inv_hbm.at[row])


def kernel(x):
    pmat = (lax.broadcasted_iota(jnp.int32, (_HW, _WP), 0) // _PATCH
            == lax.broadcasted_iota(jnp.int32, (_HW, _WP), 1)
            ).astype(jnp.bfloat16)
    q = pl.pallas_call(
        _pool_body,
        grid=(_B // _G, 2),
        in_specs=[pl.BlockSpec((_G, 1, _HW, _HW), lambda b, c: (b, c, 0, 0)),
                  pl.BlockSpec((_HW, _WP), lambda b, c: (0, 0))],
        out_specs=pl.BlockSpec((_G, 1, _HP, _WP), lambda b, c: (b, c, 0, 0)),
        out_shape=jax.ShapeDtypeStruct((_B, 2, _HP, _WP), jnp.float32),
    )(x, pmat)
    qf = q.reshape(_B, 2 * _N)
    imap, inv = _build_sort_kernel()(qf)
    batch_indices = jnp.broadcast_to(
        jnp.arange(_B, dtype=jnp.int32)[:, None], (_B, _N))
    return (batch_indices, imap, inv)


# speedup vs baseline: 1.0481x; 1.0481x over previous
import functools

import jax
import jax.numpy as jnp
from jax import lax
from jax.experimental import pallas as pl
from jax.experimental.pallas import tpu as pltpu
from jax.experimental.pallas import tpu_sc as plsc

_PATCH = 32
_THRESHOLD = 0.4
_B = 64
_HW = 512
_HP = _HW // _PATCH
_WP = _HW // _PATCH
_N = _HP * _WP
_L = 16
_NW = 32
_ROWS_PER_W = _B // _NW


_G = 4


def _pool_body(x_ref, p_ref, out_ref):
    v = x_ref[...]
    v1 = jnp.max(v.reshape(_G * 2 * _HP, _PATCH, _HW), axis=1)
    ind = (v1 > _THRESHOLD).astype(jnp.bfloat16)
    res = lax.dot_general(
        ind, p_ref[...], (((1,), (0,)), ((), ())),
        preferred_element_type=jnp.float32)
    out_ref[...] = res.reshape(_G, 2, _HP, _WP)


@functools.lru_cache(maxsize=None)
def _build_sort_kernel():
    mesh = plsc.VectorSubcoreMesh(core_axis_name="c", subcore_axis_name="s")
    return functools.partial(
        pl.kernel,
        mesh=mesh,
        compiler_params=pltpu.CompilerParams(needs_layout_passes=False),
        out_type=[
            jax.ShapeDtypeStruct((_B, _N), jnp.int32),
            jax.ShapeDtypeStruct((_B, _N), jnp.int32),
        ],
        scratch_types=[
            pltpu.VMEM((2 * _N,), jnp.float32),
            pltpu.VMEM((_N,), jnp.int32),
            pltpu.VMEM((_N,), jnp.int32),
        ],
    )(_sort_body)


def _sort_body(q_hbm, imap_hbm, inv_hbm, q_v, imap_v, inv_v):
    wid = lax.axis_index("s") * 2 + lax.axis_index("c")
    half = jnp.float32(0.5)
    for rr in range(_ROWS_PER_W):
        row = wid * _ROWS_PER_W + rr
        pltpu.sync_copy(q_hbm.at[row], q_v)

        c0 = jnp.int32(0)
        c1 = jnp.int32(0)
        for i in range(_N // _L):
            a = q_v[pl.ds(i * _L, _L)]
            b = q_v[pl.ds(_N + i * _L, _L)]
            k2 = b > half
            k0 = jnp.logical_and(a > half, jnp.logical_not(k2))
            k1 = jnp.logical_not(jnp.logical_or(k0, k2))
            c0 = c0 + jnp.sum(k0.astype(jnp.int32))
            c1 = c1 + jnp.sum(k1.astype(jnp.int32))

        r0 = jnp.int32(0)
        r1 = c0
        r2 = c0 + c1
        for i in range(_N // _L):
            a = q_v[pl.ds(i * _L, _L)]
            b = q_v[pl.ds(_N + i * _L, _L)]
            k2 = b > half
            k0 = jnp.logical_and(a > half, jnp.logical_not(k2))
            k1 = jnp.logical_not(jnp.logical_or(k0, k2))
            i0 = k0.astype(jnp.int32)
            i1 = k1.astype(jnp.int32)
            i2 = k2.astype(jnp.int32)
            e0 = plsc.cumsum(i0) - i0
            e1 = plsc.cumsum(i1) - i1
            e2 = plsc.cumsum(i2) - i2
            rank = jnp.where(k0, r0 + e0, jnp.where(k2, r2 + e2, r1 + e1))
            pos = lax.iota(jnp.int32, _L) + (i * _L)
            inv_v[pl.ds(i * _L, _L)] = rank
            plsc.store_scatter(imap_v, [rank], pos)
            r0 = r0 + jnp.sum(i0)
            r1 = r1 + jnp.sum(i1)
            r2 = r2 + jnp.sum(i2)

        pltpu.sync_copy(imap_v, imap_hbm.at[row])
        pltpu.sync_copy(inv_v, inv_hbm.at[row])


def kernel(x):
    pmat = (lax.broadcasted_iota(jnp.int32, (_HW, _WP), 0) // _PATCH
            == lax.broadcasted_iota(jnp.int32, (_HW, _WP), 1)
            ).astype(jnp.bfloat16)
    q = pl.pallas_call(
        _pool_body,
        grid=(_B // _G,),
        in_specs=[pl.BlockSpec((_G, 2, _HW, _HW), lambda b: (b, 0, 0, 0)),
                  pl.BlockSpec((_HW, _WP), lambda b: (0, 0))],
        out_specs=pl.BlockSpec((_G, 2, _HP, _WP), lambda b: (b, 0, 0, 0)),
        out_shape=jax.ShapeDtypeStruct((_B, 2, _HP, _WP), jnp.float32),
    )(x, pmat)
    qf = q.reshape(_B, 2 * _N)
    imap, inv = _build_sort_kernel()(qf)
    batch_indices = jnp.broadcast_to(
        jnp.arange(_B, dtype=jnp.int32)[:, None], (_B, _N))
    return (batch_indices, imap, inv)
